# baseline (speedup 1.0000x reference)
import jax
import jax.numpy as jnp
from jax import lax
from jax.experimental import pallas as pl
from jax.experimental.pallas import tpu as pltpu
from jax.experimental.pallas import tpu_sc as plsc

TOTAL = 3276800
VOCAB = 1000000
NC = 2
NS = 16
NW = NC * NS
PER_W = TOTAL // NW
CHUNK = 51200
NCHUNK = PER_W // CHUNK


def _vocab_gather(hashes, table, out, idx_v, rows_v, sem):
    cid = lax.axis_index("c")
    sid = lax.axis_index("s")
    wid = sid * NC + cid
    base = wid * PER_W
    for i in range(NCHUNK):
        off = base + i * CHUNK
        pltpu.sync_copy(hashes.at[pl.ds(off, CHUNK)], idx_v)
        pltpu.async_copy(table.at[idx_v], rows_v, sem).wait()
        pltpu.sync_copy(rows_v, out.at[pl.ds(off, CHUNK)])


def kernel(token_hashes, start_ids, end_ids, vocab_table):
    mesh = plsc.VectorSubcoreMesh(core_axis_name="c", subcore_axis_name="s")
    gather = pl.kernel(
        _vocab_gather,
        out_type=jax.ShapeDtypeStruct((TOTAL,), jnp.float32),
        mesh=mesh,
        scratch_types=[
            pltpu.VMEM((CHUNK,), jnp.int32),
            pltpu.VMEM((CHUNK,), jnp.float32),
            pltpu.SemaphoreType.DMA,
        ],
    )
    token_ids = gather(token_hashes, vocab_table)
    return (token_ids, start_ids, end_ids)


# speedup vs baseline: 2.1682x; 2.1682x over previous
import jax
import jax.numpy as jnp
from jax import lax
from jax.experimental import pallas as pl
from jax.experimental.pallas import tpu as pltpu
from jax.experimental.pallas import tpu_sc as plsc

TOTAL = 3276800
VOCAB = 1000000
NC = 2
NS = 16
NW = NC * NS
PER_W = TOTAL // NW
CHUNK = 12800
NCHUNK = PER_W // CHUNK
VPAD = 1000064
SEG = VPAD // NS
SEG_PIECES = (CHUNK, CHUNK, CHUNK, CHUNK, SEG - 4 * CHUNK)


def _vocab_gather(hashes, table, out, table_sh,
                  idx0, idx1, rows0, rows1,
                  isem0, isem1, gsem0, gsem1, osem0, osem1):
    cid = lax.axis_index("c")
    sid = lax.axis_index("s")
    wid = sid * NC + cid
    base = wid * PER_W
    idx_v = (idx0, idx1)
    rows_v = (rows0, rows1)
    isem = (isem0, isem1)
    gsem = (gsem0, gsem1)
    osem = (osem0, osem1)

    icp = [None] * NCHUNK
    for i in range(2):
        icp[i] = pltpu.make_async_copy(
            hashes.at[pl.ds(base + i * CHUNK, CHUNK)], idx_v[i], isem[i])
        icp[i].start()

    ld = [None, None]
    st = [None, None]
    soff = 0
    for k, sz in enumerate(SEG_PIECES):
        b = k % 2
        if st[b] is not None:
            st[b].wait()
        ld[b] = pltpu.make_async_copy(
            table.at[pl.ds(sid * SEG + soff, sz)],
            rows_v[b].at[pl.ds(0, sz)], gsem[b])
        ld[b].start()
        ld[b].wait()
        st[b] = pltpu.make_async_copy(
            rows_v[b].at[pl.ds(0, sz)],
            table_sh.at[pl.ds(sid * SEG + soff, sz)], osem[b])
        st[b].start()
        soff += sz
    for b in range(2):
        if st[b] is not None:
            st[b].wait()
    plsc.subcore_barrier()

    ocp = [None] * NCHUNK
    for i in range(NCHUNK):
        b = i % 2
        off = base + i * CHUNK
        icp[i].wait()
        if i >= 2:
            ocp[i - 2].wait()
        gcp = pltpu.make_async_copy(table_sh.at[idx_v[b]], rows_v[b], gsem[b])
        gcp.start()
        gcp.wait()
        ocp[i] = pltpu.make_async_copy(
            rows_v[b], out.at[pl.ds(off, CHUNK)], osem[b])
        ocp[i].start()
        if i + 2 < NCHUNK:
            icp[i + 2] = pltpu.make_async_copy(
                hashes.at[pl.ds(base + (i + 2) * CHUNK, CHUNK)],
                idx_v[b], isem[b])
            icp[i + 2].start()
    ocp[NCHUNK - 2].wait()
    ocp[NCHUNK - 1].wait()


def kernel(token_hashes, start_ids, end_ids, vocab_table):
    table_p = jnp.pad(vocab_table, (0, VPAD - VOCAB))
    mesh = plsc.VectorSubcoreMesh(core_axis_name="c", subcore_axis_name="s")
    gather = pl.kernel(
        _vocab_gather,
        out_type=jax.ShapeDtypeStruct((TOTAL,), jnp.float32),
        mesh=mesh,
        scratch_types=[
            pltpu.VMEM_SHARED((VPAD,), jnp.float32),
            pltpu.VMEM((CHUNK,), jnp.int32),
            pltpu.VMEM((CHUNK,), jnp.int32),
            pltpu.VMEM((CHUNK,), jnp.float32),
            pltpu.VMEM((CHUNK,), jnp.float32),
            pltpu.SemaphoreType.DMA,
            pltpu.SemaphoreType.DMA,
            pltpu.SemaphoreType.DMA,
            pltpu.SemaphoreType.DMA,
            pltpu.SemaphoreType.DMA,
            pltpu.SemaphoreType.DMA,
        ],
    )
    token_ids = gather(token_hashes, table_p)
    return (token_ids, start_ids, end_ids)
